# Initial kernel scaffold; baseline (speedup 1.0000x reference)
import jax
import jax.numpy as jnp
from jax.experimental import pallas as pl


def kernel(x, edge_index, W_l1, b_l1, W_r1, W_l2, b_l2, W_r2):
    raise NotImplementedError("write your pallas kernel here")


# speedup vs baseline: 6.1257x; 6.1257x over previous
import jax
import jax.numpy as jnp
from jax import lax
from jax.experimental import pallas as pl
from jax.experimental.pallas import tpu as pltpu
from jax.experimental.pallas import tpu_sc as plsc

N = 10000
E = 320000
D = 128

NC = 2
NS = 16
NW = NC * NS
EPW = E // NW
CH = 80
NCHUNK = EPW // CH
NP = 10240
RPT = NP // NS
CW = 16

_mesh = plsc.VectorSubcoreMesh(core_axis_name="c", subcore_axis_name="s")


def _agg_body(p_hbm, src_hbm, dst_hbm, z128_hbm, acc_out,
              src_v, dst_v, rows_v, acc_s, sem):
    c = lax.axis_index("c")
    s = lax.axis_index("s")
    wid = s * NC + c

    pltpu.sync_copy(src_hbm.at[wid], src_v)
    pltpu.sync_copy(dst_hbm.at[wid], dst_v)
    pltpu.sync_copy(z128_hbm, acc_s.at[pl.ds(s * RPT, RPT)])
    plsc.subcore_barrier()

    def step(i, carry):
        pltpu.async_copy(p_hbm.at[src_v.at[i]], rows_v, sem).wait()
        pltpu.sync_copy(rows_v, acc_s.at[dst_v.at[i]], add=True)
        return carry

    lax.fori_loop(0, NCHUNK, step, 0)
    plsc.subcore_barrier()

    rows = pl.ds(s * RPT, RPT)
    pltpu.sync_copy(acc_s.at[rows], acc_out.at[c].at[rows])


_agg = pl.kernel(
    _agg_body,
    out_type=jax.ShapeDtypeStruct((NC, NP, D), jnp.float32),
    mesh=_mesh,
    scratch_types=[
        pltpu.VMEM((NCHUNK, CH), jnp.int32),
        pltpu.VMEM((NCHUNK, CH), jnp.int32),
        pltpu.VMEM((CH, D), jnp.float32),
        pltpu.VMEM_SHARED((NP, D), jnp.float32),
        pltpu.SemaphoreType.DMA,
    ],
)


def _cnt_body(dst_hbm, ones_hbm, z128_hbm, cnt_out,
              dst_v, ones_v, cnt_s, sem):
    c = lax.axis_index("c")
    s = lax.axis_index("s")
    wid = s * NC + c

    pltpu.sync_copy(dst_hbm.at[wid], dst_v)
    pltpu.sync_copy(ones_hbm, ones_v)
    pltpu.sync_copy(z128_hbm, cnt_s.at[pl.ds(s * RPT, RPT)])
    plsc.subcore_barrier()

    def step(i, carry):
        pltpu.sync_copy(ones_v, cnt_s.at[dst_v.at[i]], add=True)
        return carry

    lax.fori_loop(0, NCHUNK, step, 0)
    plsc.subcore_barrier()

    rows = pl.ds(s * RPT, RPT)
    pltpu.sync_copy(cnt_s.at[rows], cnt_out.at[c].at[rows])


_cnt = pl.kernel(
    _cnt_body,
    out_type=jax.ShapeDtypeStruct((NC, NP, D), jnp.float32),
    mesh=_mesh,
    scratch_types=[
        pltpu.VMEM((NCHUNK, CH), jnp.int32),
        pltpu.VMEM((CH, D), jnp.float32),
        pltpu.VMEM_SHARED((NP, D), jnp.float32),
        pltpu.SemaphoreType.DMA,
    ],
)

BN = 2000


def _proj_body(x_ref, wl_ref, wr_ref, p_ref, r_ref):
    x = x_ref[...]
    p_ref[...] = jnp.dot(x, wl_ref[...], preferred_element_type=jnp.float32)
    r_ref[...] = jnp.dot(x, wr_ref[...], preferred_element_type=jnp.float32)


_row_spec = pl.BlockSpec((BN, D), lambda i: (i, 0))
_w_spec = pl.BlockSpec((D, D), lambda i: (0, 0))
_b_spec = pl.BlockSpec((1, D), lambda i: (0, 0))
_cnt_spec = _row_spec


def _proj(x, wl, wr):
    return pl.pallas_call(
        _proj_body,
        grid=(N // BN,),
        in_specs=[_row_spec, _w_spec, _w_spec],
        out_specs=[_row_spec, _row_spec],
        out_shape=[jax.ShapeDtypeStruct((N, D), jnp.float32)] * 2,
    )(x, wl, wr)


def _mid_body(a0_ref, a1_ref, c0_ref, c1_ref, b_ref, r_ref, wl_ref, wr_ref,
              p2_ref, r2_ref):
    cnt = c0_ref[...][:, :1] + c1_ref[...][:, :1]
    mean = (a0_ref[...] + a1_ref[...]) / jnp.maximum(cnt, 1.0)
    h = jnp.maximum(mean + b_ref[...] + r_ref[...], 0.0)
    p2_ref[...] = jnp.dot(h, wl_ref[...], preferred_element_type=jnp.float32)
    r2_ref[...] = jnp.dot(h, wr_ref[...], preferred_element_type=jnp.float32)


def _mid(a0, a1, c0, c1, b1, r1, wl2, wr2):
    return pl.pallas_call(
        _mid_body,
        grid=(N // BN,),
        in_specs=[_row_spec, _row_spec, _cnt_spec, _cnt_spec, _b_spec,
                  _row_spec, _w_spec, _w_spec],
        out_specs=[_row_spec, _row_spec],
        out_shape=[jax.ShapeDtypeStruct((N, D), jnp.float32)] * 2,
    )(a0, a1, c0, c1, b1, r1, wl2, wr2)


def _fin_body(a0_ref, a1_ref, c0_ref, c1_ref, b_ref, r_ref, o_ref):
    cnt = c0_ref[...][:, :1] + c1_ref[...][:, :1]
    mean = (a0_ref[...] + a1_ref[...]) / jnp.maximum(cnt, 1.0)
    o_ref[...] = mean + b_ref[...] + r_ref[...]


def _fin(a0, a1, c0, c1, b2, r2):
    return pl.pallas_call(
        _fin_body,
        grid=(N // BN,),
        in_specs=[_row_spec, _row_spec, _cnt_spec, _cnt_spec, _b_spec,
                  _row_spec],
        out_specs=_row_spec,
        out_shape=jax.ShapeDtypeStruct((N, D), jnp.float32),
    )(a0, a1, c0, c1, b2, r2)


@jax.jit
def kernel(x, edge_index, W_l1, b_l1, W_r1, W_l2, b_l2, W_r2):
    src = edge_index[0].reshape(NW, NCHUNK, CH)
    dst = edge_index[1].reshape(NW, NCHUNK, CH)
    ones = jnp.ones((CH, D), jnp.float32)
    z128 = jnp.zeros((RPT, D), jnp.float32)
    b1 = b_l1.reshape(1, D)
    b2 = b_l2.reshape(1, D)

    cnt = _cnt(dst, ones, z128)
    p1, r1 = _proj(x, W_l1, W_r1)
    acc1 = _agg(p1, src, dst, z128)
    p2, r2 = _mid(acc1[0], acc1[1], cnt[0], cnt[1], b1, r1, W_l2, W_r2)
    acc2 = _agg(p2, src, dst, z128)
    return _fin(acc2[0], acc2[1], cnt[0], cnt[1], b2, r2)
